# baseline (speedup 1.0000x reference)
import functools

import jax
import jax.numpy as jnp
from jax import lax
from jax.experimental import pallas as pl
from jax.experimental.pallas import tpu as pltpu
from jax.experimental.pallas import tpu_sc as plsc

NUM_CLASSES = 10
NC = 2
NS = 16
L = 16
NW = NC * NS
CHUNK = 16384


def _mesh():
    return plsc.VectorSubcoreMesh(core_axis_name="c", subcore_axis_name="s")


def _wid():
    return lax.axis_index("s") * NC + lax.axis_index("c")


def _minmax_pass_tc(n):
    GRID_MM = 4
    rows = n // 128 // GRID_MM

    def body(x_ref, out_ref):
        xb = x_ref[...].reshape(4, rows // 4, 128)
        out_ref[0:4, :] = jnp.min(xb, axis=1)
        out_ref[4:8, :] = jnp.max(xb, axis=1)

    def run(x):
        xr = x.reshape(GRID_MM * rows, 128)
        return pl.pallas_call(
            body,
            grid=(GRID_MM,),
            in_specs=[pl.BlockSpec((rows, 128), lambda i: (i, 0))],
            out_specs=pl.BlockSpec((8, 128), lambda i: (i, 0)),
            out_shape=jax.ShapeDtypeStruct((GRID_MM * 8, 128), jnp.float32),
        )(xr).reshape(GRID_MM, 8, 128)

    return run


def _assign_pass(n_per):
    n_chunks = n_per // CHUNK

    @functools.partial(
        pl.kernel,
        out_type=jax.ShapeDtypeStruct((n_per * NW,), jnp.int32),
        mesh=_mesh(),
        compiler_params=pltpu.CompilerParams(needs_layout_passes=False),
        scratch_types=[
            pltpu.VMEM((4, 8, 128), jnp.float32),
            pltpu.VMEM((L,), jnp.float32),
            pltpu.VMEM((L,), jnp.float32),
            pltpu.VMEM((CHUNK,), jnp.float32),
            pltpu.VMEM((CHUNK,), jnp.float32),
            pltpu.VMEM((CHUNK,), jnp.float32),
            pltpu.VMEM((CHUNK,), jnp.int32),
            pltpu.VMEM((CHUNK,), jnp.int32),
            pltpu.SemaphoreType.DMA,
            pltpu.SemaphoreType.DMA,
            pltpu.SemaphoreType.DMA,
            pltpu.SemaphoreType.DMA,
            pltpu.SemaphoreType.DMA,
        ],
    )
    def body(x_hbm, mm_hbm, out_hbm, mm_v, mn_v, mx_v, in0, in1, in2,
             o0, o1, si0, si1, si2, so0, so1):
        wid = _wid()
        base = wid * n_per
        ibufs = (in0, in1, in2)
        isems = (si0, si1, si2)
        obufs = (o0, o1)
        osems = (so0, so1)

        pltpu.sync_copy(mm_hbm, mm_v)
        mn = mm_v[0, 0, pl.ds(0, L)]
        mx = mm_v[0, 4, pl.ds(0, L)]

        def mmred(g, carry):
            m0, m1 = carry
            for r in range(4):
                for t in range(128 // L):
                    m0 = jnp.minimum(m0, mm_v[g, r, pl.ds(t * L, L)])
                    m1 = jnp.maximum(m1, mm_v[g, 4 + r, pl.ds(t * L, L)])
            return m0, m1

        mn, mx = lax.fori_loop(0, 4, mmred, (mn, mx))
        iota = lax.iota(jnp.int32, L)
        for shift in (8, 4, 2, 1):
            mn_v[:] = mn
            mx_v[:] = mx
            mn = jnp.minimum(mn, plsc.load_gather(mn_v, [iota ^ shift]))
            mx = jnp.maximum(mx, plsc.load_gather(mx_v, [iota ^ shift]))
        lo = mn + jnp.float32(1e-6)
        hi = mx - jnp.float32(1e-6)
        inv_step = jnp.float32(NUM_CLASSES - 2) / (hi - lo)
        bias = jnp.float32(0.99999988) - lo * inv_step

        in_copies = {}
        out_copies = {}
        for p in range(2):
            in_copies[p] = pltpu.async_copy(
                x_hbm.at[pl.ds(base + p * CHUNK, CHUNK)],
                ibufs[p], isems[p])

        for c in range(n_chunks):
            in_copies.pop(c).wait()
            if c + 2 < n_chunks:
                in_copies[c + 2] = pltpu.async_copy(
                    x_hbm.at[pl.ds(base + (c + 2) * CHUNK, CHUNK)],
                    ibufs[(c + 2) % 3], isems[(c + 2) % 3])
            if c >= 2:
                out_copies.pop(c - 2).wait()
            ibuf = ibufs[c % 3]
            obuf = obufs[c % 2]

            @plsc.parallel_loop(0, CHUNK, step=L, unroll=16)
            def compute(i, ibuf=ibuf, obuf=obuf):
                q = ibuf[pl.ds(i, L)] * inv_step + bias
                obuf[pl.ds(i, L)] = q.astype(jnp.int32)

            out_copies[c] = pltpu.async_copy(
                obuf, out_hbm.at[pl.ds(base + c * CHUNK, CHUNK)],
                osems[c % 2])

        for c in (n_chunks - 2, n_chunks - 1):
            if c >= 0:
                out_copies.pop(c).wait()

    return body


def kernel(input):
    n = input.shape[0]
    n_per = n // NW
    mm = _minmax_pass_tc(n)(input)
    return _assign_pass(n_per)(input, mm)


# speedup vs baseline: 1.0093x; 1.0093x over previous
import functools

import jax
import jax.numpy as jnp
from jax import lax
from jax.experimental import pallas as pl
from jax.experimental.pallas import tpu as pltpu
from jax.experimental.pallas import tpu_sc as plsc

NUM_CLASSES = 10
NC = 2
NS = 16
L = 16
NW = NC * NS
CHUNK = 16384


def _mesh():
    return plsc.VectorSubcoreMesh(core_axis_name="c", subcore_axis_name="s")


def _wid():
    return lax.axis_index("s") * NC + lax.axis_index("c")


def _minmax_pass_tc(n):
    GRID_MM = 8
    rows = n // 128 // GRID_MM

    def body(x_ref, out_ref):
        xb = x_ref[...].reshape(4, rows // 4, 128)
        out_ref[0:4, :] = jnp.min(xb, axis=1)
        out_ref[4:8, :] = jnp.max(xb, axis=1)

    def run(x):
        xr = x.reshape(GRID_MM * rows, 128)
        return pl.pallas_call(
            body,
            grid=(GRID_MM,),
            in_specs=[pl.BlockSpec((rows, 128), lambda i: (i, 0))],
            out_specs=pl.BlockSpec((8, 128), lambda i: (i, 0)),
            out_shape=jax.ShapeDtypeStruct((GRID_MM * 8, 128), jnp.float32),
        )(xr).reshape(GRID_MM, 8, 128)

    return run


def _assign_pass(n_per):
    n_chunks = n_per // CHUNK

    @functools.partial(
        pl.kernel,
        out_type=jax.ShapeDtypeStruct((n_per * NW,), jnp.int32),
        mesh=_mesh(),
        compiler_params=pltpu.CompilerParams(needs_layout_passes=False),
        scratch_types=[
            pltpu.VMEM((8, 8, 128), jnp.float32),
            pltpu.VMEM((L,), jnp.float32),
            pltpu.VMEM((L,), jnp.float32),
            pltpu.VMEM((CHUNK,), jnp.float32),
            pltpu.VMEM((CHUNK,), jnp.float32),
            pltpu.VMEM((CHUNK,), jnp.float32),
            pltpu.VMEM((CHUNK,), jnp.int32),
            pltpu.VMEM((CHUNK,), jnp.int32),
            pltpu.SemaphoreType.DMA,
            pltpu.SemaphoreType.DMA,
            pltpu.SemaphoreType.DMA,
            pltpu.SemaphoreType.DMA,
            pltpu.SemaphoreType.DMA,
        ],
    )
    def body(x_hbm, mm_hbm, out_hbm, mm_v, mn_v, mx_v, in0, in1, in2,
             o0, o1, si0, si1, si2, so0, so1):
        wid = _wid()
        base = wid * n_per
        ibufs = (in0, in1, in2)
        isems = (si0, si1, si2)
        obufs = (o0, o1)
        osems = (so0, so1)

        pltpu.sync_copy(mm_hbm, mm_v)
        mn = mm_v[0, 0, pl.ds(0, L)]
        mx = mm_v[0, 4, pl.ds(0, L)]

        def mmred(g, carry):
            m0, m1 = carry
            for r in range(4):
                for t in range(128 // L):
                    m0 = jnp.minimum(m0, mm_v[g, r, pl.ds(t * L, L)])
                    m1 = jnp.maximum(m1, mm_v[g, 4 + r, pl.ds(t * L, L)])
            return m0, m1

        mn, mx = lax.fori_loop(0, 8, mmred, (mn, mx))
        iota = lax.iota(jnp.int32, L)
        for shift in (8, 4, 2, 1):
            mn_v[:] = mn
            mx_v[:] = mx
            mn = jnp.minimum(mn, plsc.load_gather(mn_v, [iota ^ shift]))
            mx = jnp.maximum(mx, plsc.load_gather(mx_v, [iota ^ shift]))
        lo = mn + jnp.float32(1e-6)
        hi = mx - jnp.float32(1e-6)
        inv_step = jnp.float32(NUM_CLASSES - 2) / (hi - lo)
        bias = jnp.float32(0.99999988) - lo * inv_step

        in_copies = {}
        out_copies = {}
        for p in range(2):
            in_copies[p] = pltpu.async_copy(
                x_hbm.at[pl.ds(base + p * CHUNK, CHUNK)],
                ibufs[p], isems[p])

        for c in range(n_chunks):
            in_copies.pop(c).wait()
            if c + 2 < n_chunks:
                in_copies[c + 2] = pltpu.async_copy(
                    x_hbm.at[pl.ds(base + (c + 2) * CHUNK, CHUNK)],
                    ibufs[(c + 2) % 3], isems[(c + 2) % 3])
            if c >= 2:
                out_copies.pop(c - 2).wait()
            ibuf = ibufs[c % 3]
            obuf = obufs[c % 2]

            @plsc.parallel_loop(0, CHUNK, step=L, unroll=16)
            def compute(i, ibuf=ibuf, obuf=obuf):
                q = ibuf[pl.ds(i, L)] * inv_step + bias
                obuf[pl.ds(i, L)] = q.astype(jnp.int32)

            out_copies[c] = pltpu.async_copy(
                obuf, out_hbm.at[pl.ds(base + c * CHUNK, CHUNK)],
                osems[c % 2])

        for c in (n_chunks - 2, n_chunks - 1):
            if c >= 0:
                out_copies.pop(c).wait()

    return body


def kernel(input):
    n = input.shape[0]
    n_per = n // NW
    mm = _minmax_pass_tc(n)(input)
    return _assign_pass(n_per)(input, mm)
